# baseline (speedup 1.0000x reference)
import functools

import jax
import jax.numpy as jnp
from jax import lax
from jax.experimental import pallas as pl
from jax.experimental.pallas import tpu as pltpu
from jax.experimental.pallas import tpu_sc as plsc

V = 100000
D = 400
B = 16384
K = 20

DP = 512
CPR = DP // 128
NC = 2
NS = 16
NW = NC * NS
BPW = B // NW
HC = 8
NHC = BPW // HC
HK = HC * K
DJ = D // 16

_mesh = plsc.VectorSubcoreMesh(core_axis_name="c", subcore_axis_name="s")

_GDN = lax.GatherDimensionNumbers(
    offset_dims=(), collapsed_slice_dims=(0,), start_index_map=(0,))


def _lane_shuffle(a, idx):
    return lax.gather(a, idx[:, None], _GDN, slice_sizes=(1,),
                      mode=lax.GatherScatterMode.PROMISE_IN_BOUNDS)


def _hsum(a):
    iota = lax.iota(jnp.int32, 16)
    for sh in (8, 4, 2, 1):
        a = a + _lane_shuffle(a, jnp.bitwise_xor(iota, sh))
    return a


def _chunk(ref, q, j):
    return ref[q * CPR + j // 8, pl.ds((j % 8) * 16, 16)]


@functools.partial(
    pl.kernel,
    mesh=_mesh,
    compiler_params=pltpu.CompilerParams(
        needs_layout_passes=False, use_tc_tiling_on_sc=False),
    out_type=[
        jax.ShapeDtypeStruct((B,), jnp.float32),
        jax.ShapeDtypeStruct((B * K,), jnp.float32),
    ],
    scratch_types=[
        pltpu.VMEM((BPW + 8,), jnp.int32),
        pltpu.VMEM((BPW + 8,), jnp.int32),
        pltpu.VMEM((BPW * K,), jnp.int32),
        pltpu.VMEM((HC * CPR,), jnp.int32),
        pltpu.VMEM((HC * CPR,), jnp.int32),
        pltpu.VMEM((HK * CPR,), jnp.int32),
        pltpu.VMEM((HC * CPR, 128), jnp.float32),
        pltpu.VMEM((HC * CPR, 128), jnp.float32),
        pltpu.VMEM((HK * CPR, 128), jnp.float32),
        pltpu.VMEM((BPW,), jnp.float32),
        pltpu.VMEM((BPW * K,), jnp.float32),
        pltpu.SemaphoreType.DMA,
    ],
)
def _sc_scores(cw_hbm, xw_hbm, nw_hbm, in_hbm, out_hbm,
               pos_hbm, negs_hbm,
               cidx, xidx, nidx, cg, xg, ng, cen, ctx, neg,
               posb, negb, sem):
    wid = lax.axis_index("s") * NC + lax.axis_index("c")
    base = wid * BPW
    iota = lax.iota(jnp.int32, 16)
    zeros = jnp.zeros((16,), jnp.int32)

    pltpu.sync_copy(cw_hbm.at[pl.ds(base, BPW)], cidx.at[pl.ds(0, BPW)])
    pltpu.sync_copy(xw_hbm.at[pl.ds(base, BPW)], xidx.at[pl.ds(0, BPW)])
    pltpu.sync_copy(nw_hbm.at[pl.ds(base * K, BPW * K)], nidx)

    def step(s, carry):
        cvals = cidx[pl.ds(s * HC, 16)] * CPR
        xvals = xidx[pl.ds(s * HC, 16)] * CPR
        for c in range(CPR):
            plsc.store_scatter(cg, [iota * CPR + c], cvals + c,
                               mask=iota < HC)
            plsc.store_scatter(xg, [iota * CPR + c], xvals + c,
                               mask=iota < HC)
        for v in range(HK // 16):
            nvals = nidx[pl.ds(s * HK + v * 16, 16)] * CPR
            for c in range(CPR):
                plsc.store_scatter(ng, [v * 64 + iota * CPR + c], nvals + c)

        copies = [
            pltpu.async_copy(in_hbm.at[cg], cen, sem),
            pltpu.async_copy(out_hbm.at[xg], ctx, sem),
        ]
        for c in range(HK * CPR // 128):
            copies.append(pltpu.async_copy(
                out_hbm.at[ng.at[pl.ds(c * 128, 128)]],
                neg.at[pl.ds(c * 128, 128)], sem))
        for cp in copies:
            cp.wait()

        def row(r, rcarry):
            g = s * HC + r
            cvec = [_chunk(cen, r, j) for j in range(DJ)]
            acc = cvec[0] * _chunk(ctx, r, 0)
            for j in range(1, DJ):
                acc = acc + cvec[j] * _chunk(ctx, r, j)
            plsc.store_scatter(posb, [zeros + g], _hsum(acc),
                               mask=iota == 0)

            vecA = jnp.zeros((16,), jnp.float32)
            vecB = jnp.zeros((16,), jnp.float32)
            for k in range(K):
                q = r * K + k
                a = cvec[0] * _chunk(neg, q, 0)
                for j in range(1, DJ):
                    a = a + cvec[j] * _chunk(neg, q, j)
                sv = _hsum(a)
                if k < 16:
                    vecA = jnp.where(iota == k, sv, vecA)
                else:
                    vecB = jnp.where(iota == (k - 16), sv, vecB)
            plsc.store_scatter(negb, [g * K + iota], vecA)
            plsc.store_scatter(negb, [g * K + 16 + iota], vecB,
                               mask=iota < (K - 16))
            return rcarry

        lax.fori_loop(0, HC, row, 0)
        return carry

    lax.fori_loop(0, NHC, step, 0)

    pltpu.sync_copy(posb, pos_hbm.at[pl.ds(base, BPW)])
    pltpu.sync_copy(negb, negs_hbm.at[pl.ds(base * K, BPW * K)])


def _loss_body(pos_ref, neg_ref, out_ref):
    p = pos_ref[...]
    n = neg_ref[...]

    def softplus(z):
        return jnp.maximum(z, 0.0) + jnp.log(1.0 + jnp.exp(-jnp.abs(z)))

    total = jnp.sum(softplus(-p)) + jnp.sum(softplus(n))
    out_ref[0, 0] = total / B


def kernel(center_words, context_words, neg_samples, in_embed, out_embed):
    cw = center_words.astype(jnp.int32)
    xw = context_words.astype(jnp.int32)
    nw = neg_samples.astype(jnp.int32).reshape(B * K)
    in_pk = jnp.pad(in_embed, ((0, 0), (0, DP - D))).reshape(V * CPR, 128)
    out_pk = jnp.pad(out_embed, ((0, 0), (0, DP - D))).reshape(V * CPR, 128)
    pos, negs = _sc_scores(cw, xw, nw, in_pk, out_pk)
    loss = pl.pallas_call(
        _loss_body,
        out_shape=jax.ShapeDtypeStruct((1, 1), jnp.float32),
        out_specs=pl.BlockSpec(memory_space=pltpu.SMEM),
    )(pos.reshape(128, 128), negs.reshape(2560, 128))
    return loss[0, 0]


# speedup vs baseline: 1.0114x; 1.0114x over previous
import functools

import jax
import jax.numpy as jnp
from jax import lax
from jax.experimental import pallas as pl
from jax.experimental.pallas import tpu as pltpu
from jax.experimental.pallas import tpu_sc as plsc

V = 100000
D = 400
B = 16384
K = 20

DP = 512
CPR = DP // 128
NC = 2
NS = 16
NW = NC * NS
BPW = B // NW
HC = 8
NHC = BPW // HC
HK = HC * K
DJ = D // 16

RB = 400

_mesh = plsc.VectorSubcoreMesh(core_axis_name="c", subcore_axis_name="s")

_GDN = lax.GatherDimensionNumbers(
    offset_dims=(), collapsed_slice_dims=(0,), start_index_map=(0,))


def _lane_shuffle(a, idx):
    return lax.gather(a, idx[:, None], _GDN, slice_sizes=(1,),
                      mode=lax.GatherScatterMode.PROMISE_IN_BOUNDS)


def _hsum(a):
    iota = lax.iota(jnp.int32, 16)
    for sh in (8, 4, 2, 1):
        a = a + _lane_shuffle(a, jnp.bitwise_xor(iota, sh))
    return a


def _chunk(ref, q, j):
    return ref[q * CPR + j // 8, pl.ds((j % 8) * 16, 16)]


@functools.partial(
    pl.kernel,
    mesh=_mesh,
    compiler_params=pltpu.CompilerParams(
        needs_layout_passes=False, use_tc_tiling_on_sc=False),
    out_type=[
        jax.ShapeDtypeStruct((B,), jnp.float32),
        jax.ShapeDtypeStruct((B * K,), jnp.float32),
    ],
    scratch_types=[
        pltpu.VMEM((BPW + 8,), jnp.int32),
        pltpu.VMEM((BPW + 8,), jnp.int32),
        pltpu.VMEM((BPW * K,), jnp.int32),
        pltpu.VMEM((HC * CPR,), jnp.int32),
        pltpu.VMEM((HC * CPR,), jnp.int32),
        pltpu.VMEM((HK * CPR,), jnp.int32),
        pltpu.VMEM((HC * CPR, 128), jnp.float32),
        pltpu.VMEM((HC * CPR, 128), jnp.float32),
        pltpu.VMEM((HK * CPR, 128), jnp.float32),
        pltpu.VMEM((BPW,), jnp.float32),
        pltpu.VMEM((BPW * K,), jnp.float32),
        pltpu.SemaphoreType.DMA,
    ],
)
def _sc_scores(cw_hbm, xw_hbm, nw_hbm, in_hbm, out_hbm,
               pos_hbm, negs_hbm,
               cidx, xidx, nidx, cg, xg, ng, cen, ctx, neg,
               posb, negb, sem):
    wid = lax.axis_index("s") * NC + lax.axis_index("c")
    base = wid * BPW
    iota = lax.iota(jnp.int32, 16)
    zeros = jnp.zeros((16,), jnp.int32)

    pltpu.sync_copy(cw_hbm.at[pl.ds(base, BPW)], cidx.at[pl.ds(0, BPW)])
    pltpu.sync_copy(xw_hbm.at[pl.ds(base, BPW)], xidx.at[pl.ds(0, BPW)])
    pltpu.sync_copy(nw_hbm.at[pl.ds(base * K, BPW * K)], nidx)

    def cbase(r):
        return jnp.bitwise_and(r, -8) * CPR + jnp.bitwise_and(r, 7)

    def step(s, carry):
        cvals = cbase(cidx[pl.ds(s * HC, 16)])
        xvals = cbase(xidx[pl.ds(s * HC, 16)])
        for c in range(CPR):
            plsc.store_scatter(cg, [iota * CPR + c], cvals + 8 * c,
                               mask=iota < HC)
            plsc.store_scatter(xg, [iota * CPR + c], xvals + 8 * c,
                               mask=iota < HC)
        for v in range(HK // 16):
            nvals = cbase(nidx[pl.ds(s * HK + v * 16, 16)])
            for c in range(CPR):
                plsc.store_scatter(ng, [v * 64 + iota * CPR + c],
                                   nvals + 8 * c)

        copies = [
            pltpu.async_copy(in_hbm.at[cg], cen, sem),
            pltpu.async_copy(out_hbm.at[xg], ctx, sem),
        ]
        for c in range(HK * CPR // 128):
            copies.append(pltpu.async_copy(
                out_hbm.at[ng.at[pl.ds(c * 128, 128)]],
                neg.at[pl.ds(c * 128, 128)], sem))
        for cp in copies:
            cp.wait()

        def row(r, rcarry):
            g = s * HC + r
            cvec = [_chunk(cen, r, j) for j in range(DJ)]
            acc = cvec[0] * _chunk(ctx, r, 0)
            for j in range(1, DJ):
                acc = acc + cvec[j] * _chunk(ctx, r, j)
            plsc.store_scatter(posb, [zeros + g], _hsum(acc),
                               mask=iota == 0)

            vecA = jnp.zeros((16,), jnp.float32)
            vecB = jnp.zeros((16,), jnp.float32)
            for k in range(K):
                q = r * K + k
                a = cvec[0] * _chunk(neg, q, 0)
                for j in range(1, DJ):
                    a = a + cvec[j] * _chunk(neg, q, j)
                sv = _hsum(a)
                if k < 16:
                    vecA = jnp.where(iota == k, sv, vecA)
                else:
                    vecB = jnp.where(iota == (k - 16), sv, vecB)
            plsc.store_scatter(negb, [g * K + iota], vecA)
            plsc.store_scatter(negb, [g * K + 16 + iota], vecB,
                               mask=iota < (K - 16))
            return rcarry

        lax.fori_loop(0, HC, row, 0)
        return carry

    lax.fori_loop(0, NHC, step, 0)

    pltpu.sync_copy(posb, pos_hbm.at[pl.ds(base, BPW)])
    pltpu.sync_copy(negb, negs_hbm.at[pl.ds(base * K, BPW * K)])


def _repack_body(a_ref, b_ref, oa_ref, ob_ref):
    for src, dst in ((a_ref, oa_ref), (b_ref, ob_ref)):
        x3 = src[...]
        for a in range(RB // 8):
            for c in range(CPR - 1):
                dst[pl.ds((a * CPR + c) * 8, 8), :] = \
                    x3[a, :, 128 * c:128 * (c + 1)]
            dst[pl.ds((a * CPR + CPR - 1) * 8, 8), pl.ds(0, 16)] = \
                x3[a, :, 384:400]


def _repack(in_embed, out_embed):
    grid = V // RB
    ins = [x.reshape(V // 8, 8, D) for x in (in_embed, out_embed)]
    in_spec = pl.BlockSpec((RB // 8, 8, D), lambda i: (i, 0, 0))
    out_spec = pl.BlockSpec((RB * CPR, 128), lambda i: (i, 0))
    return pl.pallas_call(
        _repack_body,
        grid=(grid,),
        in_specs=[in_spec, in_spec],
        out_specs=[out_spec, out_spec],
        out_shape=[jax.ShapeDtypeStruct((V * CPR, 128), jnp.float32)] * 2,
    )(*ins)


def _loss_body(pos_ref, neg_ref, out_ref):
    p = pos_ref[...]
    n = neg_ref[...]

    def softplus(z):
        return jnp.maximum(z, 0.0) + jnp.log(1.0 + jnp.exp(-jnp.abs(z)))

    total = jnp.sum(softplus(-p)) + jnp.sum(softplus(n))
    out_ref[0, 0] = total / B


def kernel(center_words, context_words, neg_samples, in_embed, out_embed):
    cw = center_words.astype(jnp.int32)
    xw = context_words.astype(jnp.int32)
    nw = neg_samples.astype(jnp.int32).reshape(B * K)
    in_pk, out_pk = _repack(in_embed, out_embed)
    pos, negs = _sc_scores(cw, xw, nw, in_pk, out_pk)
    loss = pl.pallas_call(
        _loss_body,
        out_shape=jax.ShapeDtypeStruct((1, 1), jnp.float32),
        out_specs=pl.BlockSpec(memory_space=pltpu.SMEM),
    )(pos.reshape(128, 128), negs.reshape(2560, 128))
    return loss[0, 0]
